# baseline (speedup 1.0000x reference)
import functools

import jax
import jax.numpy as jnp
from jax import lax
from jax.experimental import pallas as pl
from jax.experimental.pallas import tpu as pltpu
from jax.experimental.pallas import tpu_sc as plsc

N = 32768
NB = 8
T = 2048
D = 128
K = 64

BLK = 512
NBLKS = N // BLK


def _knn_body(pos_ref, tmpl_ref, w1_ref, b1_ref, w2_ref, b2_ref,
              nn_ref, s_ref, x_ref, xt_ref, xt_s, xtn_s):
    @pl.when(pl.program_id(0) == 0)
    def _():
        h = jnp.maximum(
            jnp.dot(tmpl_ref[...], w1_ref[...],
                    preferred_element_type=jnp.float32) + b1_ref[...], 0.0)
        xt = jnp.dot(h, w2_ref[...],
                     preferred_element_type=jnp.float32) + b2_ref[...]
        xt_s[...] = xt
        xtn_s[...] = lax.dot_general(
            jnp.ones((1, D), jnp.float32), xt * xt,
            (((1,), (1,)), ((), ())), preferred_element_type=jnp.float32)

    h = jnp.maximum(
        jnp.dot(pos_ref[...], w1_ref[...],
                preferred_element_type=jnp.float32) + b1_ref[...], 0.0)
    x = jnp.dot(h, w2_ref[...],
                preferred_element_type=jnp.float32) + b2_ref[...]
    xn = jnp.sum(x * x, axis=1, keepdims=True)
    d2 = xtn_s[...] + lax.dot_general(-2.0 * x, xt_s[...],
                                      (((1,), (1,)), ((), ())),
                                      preferred_element_type=jnp.float32)
    m = jnp.min(d2, axis=1, keepdims=True)
    ii = lax.broadcasted_iota(jnp.int32, (BLK, T), 1)
    idx = jnp.min(jnp.where(d2 <= m, ii, T), axis=1, keepdims=True)
    nn_ref[...] = idx[None]
    s_ref[...] = (1.0 / (1.0 + (xn + m)))[None]
    x_ref[...] = x
    @pl.when(pl.program_id(0) == 0)
    def _():
        xt_ref[...] = xt_s[...]


def _knn(pos, template, W1, b1r, W2, b2r):
    return pl.pallas_call(
        _knn_body,
        grid=(NBLKS,),
        in_specs=[
            pl.BlockSpec((BLK, 3), lambda i: (i, 0)),
            pl.BlockSpec((T, 3), lambda i: (0, 0)),
            pl.BlockSpec((3, D), lambda i: (0, 0)),
            pl.BlockSpec((1, D), lambda i: (0, 0)),
            pl.BlockSpec((D, D), lambda i: (0, 0)),
            pl.BlockSpec((1, D), lambda i: (0, 0)),
        ],
        out_specs=[
            pl.BlockSpec((1, BLK, 1), lambda i: (i, 0, 0)),
            pl.BlockSpec((1, BLK, 1), lambda i: (i, 0, 0)),
            pl.BlockSpec((BLK, D), lambda i: (i, 0)),
            pl.BlockSpec((T, D), lambda i: (0, 0)),
        ],
        out_shape=[
            jax.ShapeDtypeStruct((NBLKS, BLK, 1), jnp.int32),
            jax.ShapeDtypeStruct((NBLKS, BLK, 1), jnp.float32),
            jax.ShapeDtypeStruct((N, D), jnp.float32),
            jax.ShapeDtypeStruct((T, D), jnp.float32),
        ],
        scratch_shapes=[pltpu.VMEM((T, D), jnp.float32),
                        pltpu.VMEM((1, T), jnp.float32)],
    )(pos, template, W1, b1r, W2, b2r)


_NC = 2
_NS = 16
_NW = _NC * _NS


@functools.lru_cache(maxsize=1)
def _sc_mesh():
    return plsc.VectorSubcoreMesh(core_axis_name="c", subcore_axis_name="s")


C = 128
CH = N // 128


def _topk_body(s_ref, b_ref, out_ref, keys_s):
    ri = lax.broadcasted_iota(jnp.int32, (NB, CH, 128), 1)
    li = lax.broadcasted_iota(jnp.int32, (NB, CH, 128), 2)
    b3 = lax.broadcasted_iota(jnp.int32, (NB, CH, 128), 0)
    gidx = ri * 128 + li
    sv = jnp.broadcast_to(s_ref[...][None], (NB, CH, 128))
    bv = jnp.broadcast_to(b_ref[...][None], (NB, CH, 128))
    keys_s[...] = jnp.where(bv == b3, sv, -(gidx.astype(jnp.float32) + 1.0))

    ci256 = lax.broadcasted_iota(jnp.int32, (NB, CH), 1)
    ri8 = lax.broadcasted_iota(jnp.int32, (NB, CH), 0)
    bcol = lax.broadcasted_iota(jnp.int32, (NB, 1), 0)
    kiota = lax.broadcasted_iota(jnp.int32, (NB, C), 1)
    li128 = lax.broadcasted_iota(jnp.int32, (1, 128), 1)

    cm0 = jnp.max(keys_s[...], axis=2)

    def round_(k, carry):
        cm, acc = carry
        m = jnp.max(cm, axis=1, keepdims=True)
        ch = jnp.min(jnp.where(cm >= m, ci256, CH), axis=1, keepdims=True)
        gcol = jnp.zeros((NB, 1), jnp.int32)
        for b in range(NB):
            cb = ch[b, 0]
            row = keys_s[b, pl.ds(cb, 1), :]
            mrow = jnp.max(row)
            l = jnp.min(jnp.where(row >= mrow, li128, 128))
            nrow = jnp.where(li128 == l, -3e9, row)
            keys_s[b, pl.ds(cb, 1), :] = nrow
            cm = jnp.where((ri8 == b) & (ci256 == cb), jnp.max(nrow), cm)
            gcol = jnp.where(bcol == b, cb * 128 + l, gcol)
        acc = jnp.where(kiota == k, jnp.broadcast_to(gcol, (NB, C)), acc)
        return cm, acc

    _, acc = lax.fori_loop(0, C, round_,
                           (cm0, jnp.zeros((NB, C), jnp.int32)))
    out_ref[...] = acc


def _topk(s2, b2):
    return pl.pallas_call(
        _topk_body,
        in_specs=[
            pl.BlockSpec((CH, 128), lambda: (0, 0)),
            pl.BlockSpec((CH, 128), lambda: (0, 0)),
        ],
        out_specs=pl.BlockSpec((NB, C), lambda: (0, 0)),
        out_shape=jax.ShapeDtypeStruct((NB, C), jnp.int32),
        scratch_shapes=[pltpu.VMEM((NB, CH, 128), jnp.float32)],
    )(s2, b2)


def _gh_body(cand_hbm, x_hbm, xt_hbm, nn_hbm, b_hbm, posf_hbm, tmplf_hbm,
             g_hbm, h_hbm,
             cidx_v, nnc_v, bc_v, xr_v, tr_v, key_v, wk_v, wnn_v, i3_v,
             px_v, py_v, pz_v, tx_v, ty_v, tz_v, out_v, sem):
    wid = lax.axis_index("s") * _NC + lax.axis_index("c")
    lane = lax.iota(jnp.int32, 16)

    @pl.when(wid < NB)
    def _():
        b = wid
        pltpu.sync_copy(cand_hbm.at[pl.ds(b * C, C)], cidx_v)
        pltpu.async_copy(nn_hbm.at[cidx_v], nnc_v, sem).wait()
        cpx = pltpu.async_copy(x_hbm.at[cidx_v], xr_v, sem)
        cpt = pltpu.async_copy(xt_hbm.at[nnc_v], tr_v, sem)
        pltpu.async_copy(b_hbm.at[cidx_v], bc_v, sem).wait()
        cpx.wait()
        cpt.wait()

        def group(g, c2):
            d2v = jnp.zeros((16,), jnp.float32)
            for r16 in range(16):
                r = g * 16 + r16
                acc = jnp.zeros((16,), jnp.float32)
                for j in range(D // 16):
                    dd = tr_v[r, pl.ds(j * 16, 16)] - xr_v[r, pl.ds(j * 16, 16)]
                    acc = acc + dd * dd
                d2v = jnp.where(lane == r16, jnp.sum(acc), d2v)
            sl = pl.ds(g * 16, 16)
            inb = bc_v[sl] == b
            cid = cidx_v[sl]
            key_v[sl] = jnp.where(inb, 1.0 / (1.0 + d2v),
                                  -(cid.astype(jnp.float32) + 1.0))
            return c2

        lax.fori_loop(0, C // 16, group, 0)

        kv = [key_v[pl.ds(v * 16, 16)] for v in range(C // 16)]
        cid_regs = [cidx_v[pl.ds(v * 16, 16)] for v in range(C // 16)]
        nnc_regs = [nnc_v[pl.ds(v * 16, 16)] for v in range(C // 16)]

        def round_(kk, carry):
            kv0, kv1, kv2, kv3, kv4, kv5, kv6, kv7, \
                wa0, wa1, wa2, wa3, wn0, wn1, wn2, wn3 = carry
            kvs = [kv0, kv1, kv2, kv3, kv4, kv5, kv6, kv7]
            m = kvs[0]
            for v in range(1, C // 16):
                m = jnp.maximum(m, kvs[v])
            smax = jnp.max(m)
            cstar = jnp.int32(C)
            for v in range(C // 16):
                pos_v = jnp.where(kvs[v] >= smax, lane + v * 16, C)
                cstar = jnp.minimum(cstar, jnp.min(pos_v))
            wiv = jnp.zeros((16,), jnp.int32)
            wnv = jnp.zeros((16,), jnp.int32)
            for v in range(C // 16):
                hit = lane + v * 16 == cstar
                wiv = jnp.where(hit, cid_regs[v], wiv)
                wnv = jnp.where(hit, nnc_regs[v], wnv)
            widx = jnp.sum(wiv)
            wnn = jnp.sum(wnv)
            kvs = [jnp.where(lane + v * 16 == cstar, -3e9, kvs[v])
                   for v in range(C // 16)]
            was = [wa0, wa1, wa2, wa3]
            wns = [wn0, wn1, wn2, wn3]
            was = [jnp.where(lane + w * 16 == kk, widx, was[w])
                   for w in range(K // 16)]
            wns = [jnp.where(lane + w * 16 == kk, wnn, wns[w])
                   for w in range(K // 16)]
            return tuple(kvs) + tuple(was) + tuple(wns)

        zi = jnp.zeros((16,), jnp.int32)
        res = lax.fori_loop(0, K, round_,
                            tuple(kv) + (zi, zi, zi, zi, zi, zi, zi, zi))
        for w in range(K // 16):
            wk_v[pl.ds(w * 16, 16)] = res[C // 16 + w]
            wnn_v[pl.ds(w * 16, 16)] = res[C // 16 + K // 16 + w]

        for c, dest in ((0, px_v), (1, py_v), (2, pz_v)):
            for v in range(K // 16):
                sl = pl.ds(v * 16, 16)
                i3_v[sl] = wk_v[sl] * 3 + c
            pltpu.async_copy(posf_hbm.at[i3_v], dest, sem).wait()
        for c, dest in ((0, tx_v), (1, ty_v), (2, tz_v)):
            for v in range(K // 16):
                sl = pl.ds(v * 16, 16)
                i3_v[sl] = wnn_v[sl] * 3 + c
            pltpu.async_copy(tmplf_hbm.at[i3_v], dest, sem).wait()

        ones = jnp.ones((16,), jnp.float32)

        def dotsum(ar, br):
            t = jnp.zeros((16,), jnp.float32)
            for v in range(K // 16):
                sl = pl.ds(v * 16, 16)
                av = ar[sl] if ar is not None else ones
                bv = br[sl] if br is not None else ones
                t = t + av * bv
            return jnp.sum(t)

        cm = (px_v, py_v, pz_v, None)
        cf = (tx_v, ty_v, tz_v, None)
        gvec = jnp.zeros((16,), jnp.float32)
        hvec = jnp.zeros((16,), jnp.float32)
        for i in range(4):
            for j in range(4):
                gvec = jnp.where(lane == i * 4 + j, dotsum(cm[i], cm[j]), gvec)
                hvec = jnp.where(lane == i * 4 + j, dotsum(cm[i], cf[j]), hvec)
        out_v[...] = gvec
        pltpu.sync_copy(out_v, g_hbm.at[b])
        out_v[...] = hvec
        pltpu.sync_copy(out_v, h_hbm.at[b])


@functools.lru_cache(maxsize=1)
def _gh_call():
    return functools.partial(
        pl.kernel,
        out_type=[
            jax.ShapeDtypeStruct((NB, 16), jnp.float32),
            jax.ShapeDtypeStruct((NB, 16), jnp.float32),
        ],
        scratch_types=[
            pltpu.VMEM((C,), jnp.int32),
            pltpu.VMEM((C,), jnp.int32),
            pltpu.VMEM((C,), jnp.int32),
            pltpu.VMEM((C, D), jnp.float32),
            pltpu.VMEM((C, D), jnp.float32),
            pltpu.VMEM((C,), jnp.float32),
            pltpu.VMEM((K,), jnp.int32),
            pltpu.VMEM((K,), jnp.int32),
            pltpu.VMEM((K,), jnp.int32),
            pltpu.VMEM((K,), jnp.float32),
            pltpu.VMEM((K,), jnp.float32),
            pltpu.VMEM((K,), jnp.float32),
            pltpu.VMEM((K,), jnp.float32),
            pltpu.VMEM((K,), jnp.float32),
            pltpu.VMEM((K,), jnp.float32),
            pltpu.VMEM((16,), jnp.float32),
            pltpu.SemaphoreType.DMA,
        ],
        mesh=_sc_mesh(),
        compiler_params=pltpu.CompilerParams(needs_layout_passes=False),
    )(_gh_body)


NR = N // 128


def _solve_body(g_ref, h_ref, p3_ref, bbm_ref, out_ref):
    g = g_ref[...]
    h = h_ref[...]

    def c(mat, i, j):
        k = i * 4 + j
        return mat[:, k:k + 1]

    d0 = c(g, 0, 0)
    L10 = c(g, 1, 0) / d0
    L20 = c(g, 2, 0) / d0
    L30 = c(g, 3, 0) / d0
    d1 = c(g, 1, 1) - L10 * L10 * d0
    L21 = (c(g, 2, 1) - L20 * L10 * d0) / d1
    L31 = (c(g, 3, 1) - L30 * L10 * d0) / d1
    d2_ = c(g, 2, 2) - L20 * L20 * d0 - L21 * L21 * d1
    L32 = (c(g, 3, 2) - L30 * L20 * d0 - L31 * L21 * d1) / d2_
    d3 = (c(g, 3, 3) - L30 * L30 * d0 - L31 * L31 * d1 - L32 * L32 * d2_)

    acols = []
    for j in range(4):
        h0, h1, h2, h3 = c(h, 0, j), c(h, 1, j), c(h, 2, j), c(h, 3, j)
        y0 = h0
        y1 = h1 - L10 * y0
        y2 = h2 - L20 * y0 - L21 * y1
        y3 = h3 - L30 * y0 - L31 * y1 - L32 * y2
        z0, z1, z2, z3 = y0 / d0, y1 / d1, y2 / d2_, y3 / d3
        x3 = z3
        x2 = z2 - L32 * x3
        x1 = z1 - L21 * x2 - L31 * x3
        x0 = z0 - L10 * x1 - L20 * x2 - L30 * x3
        acols.append((x0, x1, x2, x3))

    px = p3_ref[0]
    py = p3_ref[1]
    pz = p3_ref[2]
    bbm = bbm_ref[...]
    outs = []
    for j in range(3):
        acc = jnp.zeros((NR, 128), jnp.float32)
        for b in range(NB):
            a0 = acols[j][0][b, 0]
            a1 = acols[j][1][b, 0]
            a2 = acols[j][2][b, 0]
            a3 = acols[j][3][b, 0]
            val = px * a0 + py * a1 + pz * a2 + a3
            acc = jnp.where(bbm == b, val, acc)
        outs.append(acc)
    out_ref[...] = jnp.stack(outs, axis=0)


def _solve(G, H, p3, bbm):
    return pl.pallas_call(
        _solve_body,
        in_specs=[
            pl.BlockSpec((NB, 16), lambda: (0, 0)),
            pl.BlockSpec((NB, 16), lambda: (0, 0)),
            pl.BlockSpec((3, NR, 128), lambda: (0, 0, 0)),
            pl.BlockSpec((NR, 128), lambda: (0, 0)),
        ],
        out_specs=pl.BlockSpec((3, NR, 128), lambda: (0, 0, 0)),
        out_shape=jax.ShapeDtypeStruct((3, NR, 128), jnp.float32),
    )(G, H, p3, bbm)


def kernel(pos, batch, template, W1, b1, W2, b2):
    nn3, s3, X, Xt = _knn(pos, template, W1, b1.reshape(1, D), W2,
                          b2.reshape(1, D))
    nn = nn3.reshape(N)
    cand = _topk(s3.reshape(CH, 128), batch.reshape(CH, 128))
    G, H = _gh_call()(cand.reshape(NB * C), X, Xt, nn, batch,
                      pos.reshape(N * 3), template.reshape(T * 3))
    p3 = pos.T.reshape(3, NR, 128)
    out3 = _solve(G, H, p3, batch.reshape(NR, 128))
    return out3.reshape(3, N).T


# speedup vs baseline: 2.5778x; 2.5778x over previous
import functools

import jax
import jax.numpy as jnp
from jax import lax
from jax.experimental import pallas as pl
from jax.experimental.pallas import tpu as pltpu
from jax.experimental.pallas import tpu_sc as plsc

N = 32768
NB = 8
T = 2048
D = 128
K = 64

BLK = 512
NBLKS = N // BLK


def _knn_body(pos_ref, tmpl_ref, w1_ref, b1_ref, w2_ref, b2_ref,
              nn_ref, s_ref, x_ref, xt_ref, xt_s, xtn_s):
    @pl.when(pl.program_id(0) == 0)
    def _():
        h = jnp.maximum(
            jnp.dot(tmpl_ref[...], w1_ref[...],
                    preferred_element_type=jnp.float32) + b1_ref[...], 0.0)
        xt = jnp.dot(h, w2_ref[...],
                     preferred_element_type=jnp.float32) + b2_ref[...]
        xt_s[...] = xt
        xtn_s[...] = lax.dot_general(
            jnp.ones((1, D), jnp.float32), xt * xt,
            (((1,), (1,)), ((), ())), preferred_element_type=jnp.float32)

    h = jnp.maximum(
        jnp.dot(pos_ref[...], w1_ref[...],
                preferred_element_type=jnp.float32) + b1_ref[...], 0.0)
    x = jnp.dot(h, w2_ref[...],
                preferred_element_type=jnp.float32) + b2_ref[...]
    xn = jnp.sum(x * x, axis=1, keepdims=True)
    d2 = xtn_s[...] + lax.dot_general(-2.0 * x, xt_s[...],
                                      (((1,), (1,)), ((), ())),
                                      preferred_element_type=jnp.float32)
    m = jnp.min(d2, axis=1, keepdims=True)
    ii = lax.broadcasted_iota(jnp.int32, (BLK, T), 1)
    idx = jnp.min(jnp.where(d2 <= m, ii, T), axis=1, keepdims=True)
    nn_ref[...] = idx[None]
    s_ref[...] = (1.0 / (1.0 + (xn + m)))[None]
    x_ref[...] = x
    @pl.when(pl.program_id(0) == 0)
    def _():
        xt_ref[...] = xt_s[...]


def _knn(pos, template, W1, b1r, W2, b2r):
    return pl.pallas_call(
        _knn_body,
        grid=(NBLKS,),
        in_specs=[
            pl.BlockSpec((BLK, 3), lambda i: (i, 0)),
            pl.BlockSpec((T, 3), lambda i: (0, 0)),
            pl.BlockSpec((3, D), lambda i: (0, 0)),
            pl.BlockSpec((1, D), lambda i: (0, 0)),
            pl.BlockSpec((D, D), lambda i: (0, 0)),
            pl.BlockSpec((1, D), lambda i: (0, 0)),
        ],
        out_specs=[
            pl.BlockSpec((1, BLK, 1), lambda i: (i, 0, 0)),
            pl.BlockSpec((1, BLK, 1), lambda i: (i, 0, 0)),
            pl.BlockSpec((BLK, D), lambda i: (i, 0)),
            pl.BlockSpec((T, D), lambda i: (0, 0)),
        ],
        out_shape=[
            jax.ShapeDtypeStruct((NBLKS, BLK, 1), jnp.int32),
            jax.ShapeDtypeStruct((NBLKS, BLK, 1), jnp.float32),
            jax.ShapeDtypeStruct((N, D), jnp.float32),
            jax.ShapeDtypeStruct((T, D), jnp.float32),
        ],
        scratch_shapes=[pltpu.VMEM((T, D), jnp.float32),
                        pltpu.VMEM((1, T), jnp.float32)],
    )(pos, template, W1, b1r, W2, b2r)


_NC = 2
_NS = 16
_NW = _NC * _NS


@functools.lru_cache(maxsize=1)
def _sc_mesh():
    return plsc.VectorSubcoreMesh(core_axis_name="c", subcore_axis_name="s")


CAP = 256
CH = N // 128
NG = N // 16


def _tau_body(s_ref, b_ref, tau_ref, w_s):
    b3 = lax.broadcasted_iota(jnp.int32, (NB, CH, 128), 0)
    sv = jnp.broadcast_to(s_ref[...][None], (NB, CH, 128))
    bv = jnp.broadcast_to(b_ref[...][None], (NB, CH, 128))
    w_s[...] = jnp.where(bv == b3, sv, -1.0)

    z = jnp.zeros((NB, 1, 1), jnp.float32)
    lo, hi, tau = z, z + 1.0, z
    done = jnp.zeros((NB, 1, 1), jnp.bool_)
    w = w_s[...]
    for _ in range(10):
        mid = 0.5 * (lo + hi)
        cnt = jnp.sum(jnp.where(w > mid, 1, 0), axis=(1, 2),
                      keepdims=True)
        inw = (cnt >= K + 32) & (cnt <= CAP)
        ndone = jnp.logical_not(done)
        tau = jnp.where(ndone & inw, mid, tau)
        done = done | inw
        lo = jnp.where(ndone & (cnt > CAP), mid, lo)
        hi = jnp.where(ndone & (cnt < K), mid, hi)
    tau = jnp.where(done, tau, lo)
    l16 = lax.broadcasted_iota(jnp.int32, (1, 16), 1)
    out = jnp.zeros((1, 16), jnp.float32)
    for b in range(NB):
        out = jnp.where(l16 == b, tau[b, 0, 0], out)
    tau_ref[...] = out


def _tau(s2, b2):
    return pl.pallas_call(
        _tau_body,
        in_specs=[
            pl.BlockSpec((CH, 128), lambda: (0, 0)),
            pl.BlockSpec((CH, 128), lambda: (0, 0)),
        ],
        out_specs=pl.BlockSpec((1, 16), lambda: (0, 0)),
        out_shape=jax.ShapeDtypeStruct((1, 16), jnp.float32),
        scratch_shapes=[pltpu.VMEM((NB, CH, 128), jnp.float32)],
    )(s2, b2)


def _gh_body(s_hbm, b_hbm, tau_hbm, x_hbm, xt_hbm, nn_hbm, posf_hbm,
             tmplf_hbm, g_hbm, h_hbm,
             s_v, bt_v, tau_v, cand_v, cidx_v, nnc_v, nncf_v, xr_v, tr_v,
             key_v,
             wk_v, wnn_v, i3_v, px_v, py_v, pz_v, tx_v, ty_v, tz_v,
             out_v, sem):
    wid = lax.axis_index("s") * _NC + lax.axis_index("c")
    lane = lax.iota(jnp.int32, 16)

    @pl.when(wid < NB)
    def _():
        b = wid
        pltpu.sync_copy(s_hbm, s_v)
        pltpu.sync_copy(b_hbm, bt_v)
        pltpu.sync_copy(tau_hbm, tau_v)
        taub = jnp.max(jnp.where(lane == b, tau_v[...], -1.0))

        for v in range(CAP // 16):
            cand_v[pl.ds(v * 16, 16)] = jnp.zeros((16,), jnp.int32)
            key_v[pl.ds(v * 16, 16)] = jnp.full((16,), -3e9, jnp.float32)

        def collect(g, off):
            sl = pl.ds(g * 16, 16)
            mask = (bt_v[sl] == b) & (s_v[sl] > taub)
            mi = jnp.where(mask, 1, 0)
            pos = jnp.clip(off + plsc.cumsum(mi) - 1, 0, CAP - 1)
            plsc.store_scatter(cand_v, [pos], g * 16 + lane, mask=mask)
            return off + plsc.all_reduce_population_count(mask)

        offs = lax.fori_loop(0, NG, collect, jnp.zeros((16,), jnp.int32))
        cnt = jnp.max(offs)

        for c in range(CAP // 128):
            @pl.when(c * 128 < cnt)
            def _(c=c):
                for v in range(128 // 16):
                    cidx_v[pl.ds(v * 16, 16)] = \
                        cand_v[pl.ds(c * 128 + v * 16, 16)]
                pltpu.async_copy(nn_hbm.at[cidx_v], nnc_v, sem).wait()
                pltpu.async_copy(x_hbm.at[cidx_v], xr_v, sem).wait()
                pltpu.async_copy(xt_hbm.at[nnc_v], tr_v, sem).wait()
                for v in range(128 // 16):
                    nncf_v[pl.ds(c * 128 + v * 16, 16)] = \
                        nnc_v[pl.ds(v * 16, 16)]

                def group(g, c2):
                    d2v = jnp.zeros((16,), jnp.float32)
                    for r16 in range(16):
                        r = g * 16 + r16
                        acc = jnp.zeros((16,), jnp.float32)
                        for j in range(D // 16):
                            dd = (tr_v[r, pl.ds(j * 16, 16)]
                                  - xr_v[r, pl.ds(j * 16, 16)])
                            acc = acc + dd * dd
                        d2v = jnp.where(lane == r16, jnp.sum(acc), d2v)
                    base = c * 128 + g * 16
                    keyv = jnp.where(base + lane < cnt,
                                     1.0 / (1.0 + d2v), -3e9)
                    key_v[pl.ds(base, 16)] = keyv
                    return c2

                lax.fori_loop(0, 128 // 16, group, 0)

        kv = [key_v[pl.ds(v * 16, 16)] for v in range(CAP // 16)]
        NV = CAP // 16

        def round_(kk, carry):
            kvs = list(carry[:NV])
            wps = list(carry[NV:])
            m = kvs[0]
            for v in range(1, NV):
                m = jnp.maximum(m, kvs[v])
            smax = jnp.max(m)
            cstar = jnp.int32(CAP)
            for v in range(NV):
                pos_v = jnp.where(kvs[v] >= smax, lane + v * 16, CAP)
                cstar = jnp.minimum(cstar, jnp.min(pos_v))
            cstar = jnp.minimum(cstar, CAP - 1)
            kvs = [jnp.where(lane + v * 16 == cstar, -3e9, kvs[v])
                   for v in range(NV)]
            wps = [jnp.where(lane + w * 16 == kk, cstar, wps[w])
                   for w in range(K // 16)]
            return tuple(kvs) + tuple(wps)

        zi = jnp.zeros((16,), jnp.int32)
        res = lax.fori_loop(0, K, round_,
                            tuple(kv) + (zi,) * (K // 16))
        for w in range(K // 16):
            wpv = res[NV + w]
            wk_v[pl.ds(w * 16, 16)] = plsc.load_gather(cand_v, [wpv])
            wnn_v[pl.ds(w * 16, 16)] = plsc.load_gather(nncf_v, [wpv])

        for c, dest in ((0, px_v), (1, py_v), (2, pz_v)):
            for v in range(K // 16):
                sl = pl.ds(v * 16, 16)
                i3_v[sl] = wk_v[sl] * 3 + c
            pltpu.async_copy(posf_hbm.at[i3_v], dest, sem).wait()
        for c, dest in ((0, tx_v), (1, ty_v), (2, tz_v)):
            for v in range(K // 16):
                sl = pl.ds(v * 16, 16)
                i3_v[sl] = wnn_v[sl] * 3 + c
            pltpu.async_copy(tmplf_hbm.at[i3_v], dest, sem).wait()

        ones = jnp.ones((16,), jnp.float32)

        def dotsum(ar, br):
            t = jnp.zeros((16,), jnp.float32)
            for v in range(K // 16):
                sl = pl.ds(v * 16, 16)
                av = ar[sl] if ar is not None else ones
                bv = br[sl] if br is not None else ones
                t = t + av * bv
            return jnp.sum(t)

        cm = (px_v, py_v, pz_v, None)
        cf = (tx_v, ty_v, tz_v, None)
        gvec = jnp.zeros((16,), jnp.float32)
        hvec = jnp.zeros((16,), jnp.float32)
        for i in range(4):
            for j in range(4):
                gvec = jnp.where(lane == i * 4 + j, dotsum(cm[i], cm[j]), gvec)
                hvec = jnp.where(lane == i * 4 + j, dotsum(cm[i], cf[j]), hvec)
        out_v[...] = gvec
        pltpu.sync_copy(out_v, g_hbm.at[b])
        out_v[...] = hvec
        pltpu.sync_copy(out_v, h_hbm.at[b])


@functools.lru_cache(maxsize=1)
def _gh_call():
    return functools.partial(
        pl.kernel,
        out_type=[
            jax.ShapeDtypeStruct((NB, 16), jnp.float32),
            jax.ShapeDtypeStruct((NB, 16), jnp.float32),
        ],
        scratch_types=[
            pltpu.VMEM((N,), jnp.float32),
            pltpu.VMEM((N,), jnp.int32),
            pltpu.VMEM((16,), jnp.float32),
            pltpu.VMEM((CAP,), jnp.int32),
            pltpu.VMEM((128,), jnp.int32),
            pltpu.VMEM((128,), jnp.int32),
            pltpu.VMEM((CAP,), jnp.int32),
            pltpu.VMEM((128, D), jnp.float32),
            pltpu.VMEM((128, D), jnp.float32),
            pltpu.VMEM((CAP,), jnp.float32),
            pltpu.VMEM((K,), jnp.int32),
            pltpu.VMEM((K,), jnp.int32),
            pltpu.VMEM((K,), jnp.int32),
            pltpu.VMEM((K,), jnp.float32),
            pltpu.VMEM((K,), jnp.float32),
            pltpu.VMEM((K,), jnp.float32),
            pltpu.VMEM((K,), jnp.float32),
            pltpu.VMEM((K,), jnp.float32),
            pltpu.VMEM((K,), jnp.float32),
            pltpu.VMEM((16,), jnp.float32),
            pltpu.SemaphoreType.DMA,
        ],
        mesh=_sc_mesh(),
        compiler_params=pltpu.CompilerParams(needs_layout_passes=False),
    )(_gh_body)


NR = N // 128


def _solve_body(g_ref, h_ref, p3_ref, bbm_ref, out_ref):
    g = g_ref[...]
    h = h_ref[...]

    def c(mat, i, j):
        k = i * 4 + j
        return mat[:, k:k + 1]

    d0 = c(g, 0, 0)
    L10 = c(g, 1, 0) / d0
    L20 = c(g, 2, 0) / d0
    L30 = c(g, 3, 0) / d0
    d1 = c(g, 1, 1) - L10 * L10 * d0
    L21 = (c(g, 2, 1) - L20 * L10 * d0) / d1
    L31 = (c(g, 3, 1) - L30 * L10 * d0) / d1
    d2_ = c(g, 2, 2) - L20 * L20 * d0 - L21 * L21 * d1
    L32 = (c(g, 3, 2) - L30 * L20 * d0 - L31 * L21 * d1) / d2_
    d3 = (c(g, 3, 3) - L30 * L30 * d0 - L31 * L31 * d1 - L32 * L32 * d2_)

    acols = []
    for j in range(4):
        h0, h1, h2, h3 = c(h, 0, j), c(h, 1, j), c(h, 2, j), c(h, 3, j)
        y0 = h0
        y1 = h1 - L10 * y0
        y2 = h2 - L20 * y0 - L21 * y1
        y3 = h3 - L30 * y0 - L31 * y1 - L32 * y2
        z0, z1, z2, z3 = y0 / d0, y1 / d1, y2 / d2_, y3 / d3
        x3 = z3
        x2 = z2 - L32 * x3
        x1 = z1 - L21 * x2 - L31 * x3
        x0 = z0 - L10 * x1 - L20 * x2 - L30 * x3
        acols.append((x0, x1, x2, x3))

    px = p3_ref[0]
    py = p3_ref[1]
    pz = p3_ref[2]
    bbm = bbm_ref[...]
    outs = []
    for j in range(3):
        acc = jnp.zeros((NR, 128), jnp.float32)
        for b in range(NB):
            a0 = acols[j][0][b, 0]
            a1 = acols[j][1][b, 0]
            a2 = acols[j][2][b, 0]
            a3 = acols[j][3][b, 0]
            val = px * a0 + py * a1 + pz * a2 + a3
            acc = jnp.where(bbm == b, val, acc)
        outs.append(acc)
    out_ref[...] = jnp.stack(outs, axis=0)


def _solve(G, H, p3, bbm):
    return pl.pallas_call(
        _solve_body,
        in_specs=[
            pl.BlockSpec((NB, 16), lambda: (0, 0)),
            pl.BlockSpec((NB, 16), lambda: (0, 0)),
            pl.BlockSpec((3, NR, 128), lambda: (0, 0, 0)),
            pl.BlockSpec((NR, 128), lambda: (0, 0)),
        ],
        out_specs=pl.BlockSpec((3, NR, 128), lambda: (0, 0, 0)),
        out_shape=jax.ShapeDtypeStruct((3, NR, 128), jnp.float32),
    )(G, H, p3, bbm)


def kernel(pos, batch, template, W1, b1, W2, b2):
    nn3, s3, X, Xt = _knn(pos, template, W1, b1.reshape(1, D), W2,
                          b2.reshape(1, D))
    nn = nn3.reshape(N)
    tau = _tau(s3.reshape(CH, 128), batch.reshape(CH, 128))
    G, H = _gh_call()(s3.reshape(N), batch, tau.reshape(16), X, Xt, nn,
                      pos.reshape(N * 3), template.reshape(T * 3))
    p3 = pos.T.reshape(3, NR, 128)
    out3 = _solve(G, H, p3, batch.reshape(NR, 128))
    return out3.reshape(3, N).T


# speedup vs baseline: 2.8499x; 1.1056x over previous
import functools

import jax
import jax.numpy as jnp
from jax import lax
from jax.experimental import pallas as pl
from jax.experimental.pallas import tpu as pltpu
from jax.experimental.pallas import tpu_sc as plsc

N = 32768
NB = 8
T = 2048
D = 128
K = 64

BLK = 512
NBLKS = N // BLK


def _knn_body(pos_ref, tmpl_ref, w1_ref, b1_ref, w2_ref, b2_ref,
              nn_ref, s_ref, x_ref, xt_ref, xt_s, xtn_s):
    @pl.when(pl.program_id(0) == 0)
    def _():
        h = jnp.maximum(
            jnp.dot(tmpl_ref[...], w1_ref[...],
                    preferred_element_type=jnp.float32) + b1_ref[...], 0.0)
        xt = jnp.dot(h, w2_ref[...],
                     preferred_element_type=jnp.float32) + b2_ref[...]
        xt_s[...] = xt
        xtn_s[...] = lax.dot_general(
            jnp.ones((1, D), jnp.float32), xt * xt,
            (((1,), (1,)), ((), ())), preferred_element_type=jnp.float32)

    h = jnp.maximum(
        jnp.dot(pos_ref[...], w1_ref[...],
                preferred_element_type=jnp.float32) + b1_ref[...], 0.0)
    x = jnp.dot(h, w2_ref[...],
                preferred_element_type=jnp.float32) + b2_ref[...]
    xn = jnp.sum(x * x, axis=1, keepdims=True)
    d2 = xtn_s[...] + lax.dot_general(-2.0 * x, xt_s[...],
                                      (((1,), (1,)), ((), ())),
                                      preferred_element_type=jnp.float32)
    m = jnp.min(d2, axis=1, keepdims=True)
    ii = lax.broadcasted_iota(jnp.int32, (BLK, T), 1)
    idx = jnp.min(jnp.where(d2 <= m, ii, T), axis=1, keepdims=True)
    nn_ref[...] = idx[None]
    s_ref[...] = (1.0 / (1.0 + (xn + m)))[None]
    x_ref[...] = x
    @pl.when(pl.program_id(0) == 0)
    def _():
        xt_ref[...] = xt_s[...]


def _knn(pos, template, W1, b1r, W2, b2r):
    return pl.pallas_call(
        _knn_body,
        grid=(NBLKS,),
        in_specs=[
            pl.BlockSpec((BLK, 3), lambda i: (i, 0)),
            pl.BlockSpec((T, 3), lambda i: (0, 0)),
            pl.BlockSpec((3, D), lambda i: (0, 0)),
            pl.BlockSpec((1, D), lambda i: (0, 0)),
            pl.BlockSpec((D, D), lambda i: (0, 0)),
            pl.BlockSpec((1, D), lambda i: (0, 0)),
        ],
        out_specs=[
            pl.BlockSpec((1, BLK, 1), lambda i: (i, 0, 0)),
            pl.BlockSpec((1, BLK, 1), lambda i: (i, 0, 0)),
            pl.BlockSpec((BLK, D), lambda i: (i, 0)),
            pl.BlockSpec((T, D), lambda i: (0, 0)),
        ],
        out_shape=[
            jax.ShapeDtypeStruct((NBLKS, BLK, 1), jnp.int32),
            jax.ShapeDtypeStruct((NBLKS, BLK, 1), jnp.float32),
            jax.ShapeDtypeStruct((N, D), jnp.float32),
            jax.ShapeDtypeStruct((T, D), jnp.float32),
        ],
        scratch_shapes=[pltpu.VMEM((T, D), jnp.float32),
                        pltpu.VMEM((1, T), jnp.float32)],
    )(pos, template, W1, b1r, W2, b2r)


_NC = 2
_NS = 16
_NW = _NC * _NS


@functools.lru_cache(maxsize=1)
def _sc_mesh():
    return plsc.VectorSubcoreMesh(core_axis_name="c", subcore_axis_name="s")


CAP = 256
CH = N // 128
NG = N // 16


def _tau_body(s_ref, b_ref, tau_ref, w_s):
    b3 = lax.broadcasted_iota(jnp.int32, (NB, CH, 128), 0)
    sv = jnp.broadcast_to(s_ref[...][None], (NB, CH, 128))
    bv = jnp.broadcast_to(b_ref[...][None], (NB, CH, 128))
    w_s[...] = jnp.where(bv == b3, sv, -1.0)

    z = jnp.zeros((NB, 1, 1), jnp.float32)
    lo, hi, tau = z, z + 1.0, z
    done = jnp.zeros((NB, 1, 1), jnp.bool_)
    w = w_s[...]
    for _ in range(10):
        mid = 0.5 * (lo + hi)
        cnt = jnp.sum(jnp.where(w > mid, 1, 0), axis=(1, 2),
                      keepdims=True)
        inw = (cnt >= K + 32) & (cnt <= CAP)
        ndone = jnp.logical_not(done)
        tau = jnp.where(ndone & inw, mid, tau)
        done = done | inw
        lo = jnp.where(ndone & (cnt > CAP), mid, lo)
        hi = jnp.where(ndone & (cnt < K), mid, hi)
    tau = jnp.where(done, tau, lo)
    l16 = lax.broadcasted_iota(jnp.int32, (1, 16), 1)
    out = jnp.zeros((1, 16), jnp.float32)
    for b in range(NB):
        out = jnp.where(l16 == b, tau[b, 0, 0], out)
    tau_ref[...] = out


def _tau(s2, b2):
    return pl.pallas_call(
        _tau_body,
        in_specs=[
            pl.BlockSpec((CH, 128), lambda: (0, 0)),
            pl.BlockSpec((CH, 128), lambda: (0, 0)),
        ],
        out_specs=pl.BlockSpec((1, 16), lambda: (0, 0)),
        out_shape=jax.ShapeDtypeStruct((1, 16), jnp.float32),
        scratch_shapes=[pltpu.VMEM((NB, CH, 128), jnp.float32)],
    )(s2, b2)


def _gh_body(s_hbm, b_hbm, tau_hbm, x_hbm, xt_hbm, nn_hbm, posf_hbm,
             tmplf_hbm, g_hbm, h_hbm,
             s_v, bt_v, tau_v, cand_v, cidx_v, nnc_v, nncf_v, xr_v, tr_v,
             key_v,
             wk_v, wnn_v, i3_v, px_v, py_v, pz_v, tx_v, ty_v, tz_v,
             out_v, sem):
    wid = lax.axis_index("s") * _NC + lax.axis_index("c")
    lane = lax.iota(jnp.int32, 16)

    @pl.when(wid < NB)
    def _():
        b = wid
        pltpu.sync_copy(s_hbm, s_v)
        pltpu.sync_copy(b_hbm, bt_v)
        pltpu.sync_copy(tau_hbm, tau_v)
        taub = jnp.max(jnp.where(lane == b, tau_v[...], -1.0))

        def fg(bb):
            def step(i, c):
                lo, hi = c
                mid = jnp.minimum((lo + hi) // 2, NG - 1)
                grp = bt_v[pl.ds(mid * 16, 16)]
                last = jnp.max(jnp.where(lane == 15, grp, -1))
                pred = last >= bb
                return (jnp.where(pred, lo, mid + 1),
                        jnp.where(pred, mid, hi))
            lo, _ = lax.fori_loop(0, 11, step,
                                  (jnp.int32(0), jnp.int32(NG)))
            return lo

        g0 = fg(b)
        g1 = jnp.minimum(fg(b + 1) + 1, NG)

        for v in range(CAP // 16):
            cand_v[pl.ds(v * 16, 16)] = jnp.zeros((16,), jnp.int32)
            key_v[pl.ds(v * 16, 16)] = jnp.full((16,), -3e9, jnp.float32)

        def collect(g, off):
            sl = pl.ds(g * 16, 16)
            mask = (bt_v[sl] == b) & (s_v[sl] > taub)
            mi = jnp.where(mask, 1, 0)
            pos = jnp.clip(off + plsc.cumsum(mi) - 1, 0, CAP - 1)
            plsc.store_scatter(cand_v, [pos], g * 16 + lane, mask=mask)
            return off + plsc.all_reduce_population_count(mask)

        offs = lax.fori_loop(g0, g1, collect, jnp.zeros((16,), jnp.int32))
        cnt = jnp.max(offs)

        for c in range(CAP // 128):
            @pl.when(c * 128 < cnt)
            def _(c=c):
                for v in range(128 // 16):
                    cidx_v[pl.ds(v * 16, 16)] = \
                        cand_v[pl.ds(c * 128 + v * 16, 16)]
                pltpu.async_copy(nn_hbm.at[cidx_v], nnc_v, sem).wait()
                pltpu.async_copy(x_hbm.at[cidx_v], xr_v, sem).wait()
                pltpu.async_copy(xt_hbm.at[nnc_v], tr_v, sem).wait()
                for v in range(128 // 16):
                    nncf_v[pl.ds(c * 128 + v * 16, 16)] = \
                        nnc_v[pl.ds(v * 16, 16)]

                def group(g, c2):
                    d2v = jnp.zeros((16,), jnp.float32)
                    for r16 in range(16):
                        r = g * 16 + r16
                        acc = jnp.zeros((16,), jnp.float32)
                        for j in range(D // 16):
                            dd = (tr_v[r, pl.ds(j * 16, 16)]
                                  - xr_v[r, pl.ds(j * 16, 16)])
                            acc = acc + dd * dd
                        d2v = jnp.where(lane == r16, jnp.sum(acc), d2v)
                    base = c * 128 + g * 16
                    keyv = jnp.where(base + lane < cnt,
                                     1.0 / (1.0 + d2v), -3e9)
                    key_v[pl.ds(base, 16)] = keyv
                    return c2

                lax.fori_loop(0, 128 // 16, group, 0)

        kv = [key_v[pl.ds(v * 16, 16)] for v in range(CAP // 16)]
        NV = CAP // 16

        def round_(kk, carry):
            kvs = list(carry[:NV])
            wps = list(carry[NV:])
            m = kvs[0]
            for v in range(1, NV):
                m = jnp.maximum(m, kvs[v])
            smax = jnp.max(m)
            cstar = jnp.int32(CAP)
            for v in range(NV):
                pos_v = jnp.where(kvs[v] >= smax, lane + v * 16, CAP)
                cstar = jnp.minimum(cstar, jnp.min(pos_v))
            cstar = jnp.minimum(cstar, CAP - 1)
            kvs = [jnp.where(lane + v * 16 == cstar, -3e9, kvs[v])
                   for v in range(NV)]
            wps = [jnp.where(lane + w * 16 == kk, cstar, wps[w])
                   for w in range(K // 16)]
            return tuple(kvs) + tuple(wps)

        zi = jnp.zeros((16,), jnp.int32)
        res = lax.fori_loop(0, K, round_,
                            tuple(kv) + (zi,) * (K // 16))
        for w in range(K // 16):
            wpv = res[NV + w]
            wk_v[pl.ds(w * 16, 16)] = plsc.load_gather(cand_v, [wpv])
            wnn_v[pl.ds(w * 16, 16)] = plsc.load_gather(nncf_v, [wpv])

        for c, dest in ((0, px_v), (1, py_v), (2, pz_v)):
            for v in range(K // 16):
                sl = pl.ds(v * 16, 16)
                i3_v[sl] = wk_v[sl] * 3 + c
            pltpu.async_copy(posf_hbm.at[i3_v], dest, sem).wait()
        for c, dest in ((0, tx_v), (1, ty_v), (2, tz_v)):
            for v in range(K // 16):
                sl = pl.ds(v * 16, 16)
                i3_v[sl] = wnn_v[sl] * 3 + c
            pltpu.async_copy(tmplf_hbm.at[i3_v], dest, sem).wait()

        ones = jnp.ones((16,), jnp.float32)

        def dotsum(ar, br):
            t = jnp.zeros((16,), jnp.float32)
            for v in range(K // 16):
                sl = pl.ds(v * 16, 16)
                av = ar[sl] if ar is not None else ones
                bv = br[sl] if br is not None else ones
                t = t + av * bv
            return jnp.sum(t)

        cm = (px_v, py_v, pz_v, None)
        cf = (tx_v, ty_v, tz_v, None)
        gvec = jnp.zeros((16,), jnp.float32)
        hvec = jnp.zeros((16,), jnp.float32)
        for i in range(4):
            for j in range(4):
                gvec = jnp.where(lane == i * 4 + j, dotsum(cm[i], cm[j]), gvec)
                hvec = jnp.where(lane == i * 4 + j, dotsum(cm[i], cf[j]), hvec)
        out_v[...] = gvec
        pltpu.sync_copy(out_v, g_hbm.at[b])
        out_v[...] = hvec
        pltpu.sync_copy(out_v, h_hbm.at[b])


@functools.lru_cache(maxsize=1)
def _gh_call():
    return functools.partial(
        pl.kernel,
        out_type=[
            jax.ShapeDtypeStruct((NB, 16), jnp.float32),
            jax.ShapeDtypeStruct((NB, 16), jnp.float32),
        ],
        scratch_types=[
            pltpu.VMEM((N,), jnp.float32),
            pltpu.VMEM((N,), jnp.int32),
            pltpu.VMEM((16,), jnp.float32),
            pltpu.VMEM((CAP,), jnp.int32),
            pltpu.VMEM((128,), jnp.int32),
            pltpu.VMEM((128,), jnp.int32),
            pltpu.VMEM((CAP,), jnp.int32),
            pltpu.VMEM((128, D), jnp.float32),
            pltpu.VMEM((128, D), jnp.float32),
            pltpu.VMEM((CAP,), jnp.float32),
            pltpu.VMEM((K,), jnp.int32),
            pltpu.VMEM((K,), jnp.int32),
            pltpu.VMEM((K,), jnp.int32),
            pltpu.VMEM((K,), jnp.float32),
            pltpu.VMEM((K,), jnp.float32),
            pltpu.VMEM((K,), jnp.float32),
            pltpu.VMEM((K,), jnp.float32),
            pltpu.VMEM((K,), jnp.float32),
            pltpu.VMEM((K,), jnp.float32),
            pltpu.VMEM((16,), jnp.float32),
            pltpu.SemaphoreType.DMA,
        ],
        mesh=_sc_mesh(),
        compiler_params=pltpu.CompilerParams(needs_layout_passes=False),
    )(_gh_body)


NR = N // 128


def _solve_body(g_ref, h_ref, p3_ref, bbm_ref, out_ref):
    g = g_ref[...]
    h = h_ref[...]

    def c(mat, i, j):
        k = i * 4 + j
        return mat[:, k:k + 1]

    d0 = c(g, 0, 0)
    L10 = c(g, 1, 0) / d0
    L20 = c(g, 2, 0) / d0
    L30 = c(g, 3, 0) / d0
    d1 = c(g, 1, 1) - L10 * L10 * d0
    L21 = (c(g, 2, 1) - L20 * L10 * d0) / d1
    L31 = (c(g, 3, 1) - L30 * L10 * d0) / d1
    d2_ = c(g, 2, 2) - L20 * L20 * d0 - L21 * L21 * d1
    L32 = (c(g, 3, 2) - L30 * L20 * d0 - L31 * L21 * d1) / d2_
    d3 = (c(g, 3, 3) - L30 * L30 * d0 - L31 * L31 * d1 - L32 * L32 * d2_)

    acols = []
    for j in range(4):
        h0, h1, h2, h3 = c(h, 0, j), c(h, 1, j), c(h, 2, j), c(h, 3, j)
        y0 = h0
        y1 = h1 - L10 * y0
        y2 = h2 - L20 * y0 - L21 * y1
        y3 = h3 - L30 * y0 - L31 * y1 - L32 * y2
        z0, z1, z2, z3 = y0 / d0, y1 / d1, y2 / d2_, y3 / d3
        x3 = z3
        x2 = z2 - L32 * x3
        x1 = z1 - L21 * x2 - L31 * x3
        x0 = z0 - L10 * x1 - L20 * x2 - L30 * x3
        acols.append((x0, x1, x2, x3))

    px = p3_ref[0]
    py = p3_ref[1]
    pz = p3_ref[2]
    bbm = bbm_ref[...]
    outs = []
    for j in range(3):
        acc = jnp.zeros((NR, 128), jnp.float32)
        for b in range(NB):
            a0 = acols[j][0][b, 0]
            a1 = acols[j][1][b, 0]
            a2 = acols[j][2][b, 0]
            a3 = acols[j][3][b, 0]
            val = px * a0 + py * a1 + pz * a2 + a3
            acc = jnp.where(bbm == b, val, acc)
        outs.append(acc)
    out_ref[...] = jnp.stack(outs, axis=0)


def _solve(G, H, p3, bbm):
    return pl.pallas_call(
        _solve_body,
        in_specs=[
            pl.BlockSpec((NB, 16), lambda: (0, 0)),
            pl.BlockSpec((NB, 16), lambda: (0, 0)),
            pl.BlockSpec((3, NR, 128), lambda: (0, 0, 0)),
            pl.BlockSpec((NR, 128), lambda: (0, 0)),
        ],
        out_specs=pl.BlockSpec((3, NR, 128), lambda: (0, 0, 0)),
        out_shape=jax.ShapeDtypeStruct((3, NR, 128), jnp.float32),
    )(G, H, p3, bbm)


def kernel(pos, batch, template, W1, b1, W2, b2):
    nn3, s3, X, Xt = _knn(pos, template, W1, b1.reshape(1, D), W2,
                          b2.reshape(1, D))
    nn = nn3.reshape(N)
    tau = _tau(s3.reshape(CH, 128), batch.reshape(CH, 128))
    G, H = _gh_call()(s3.reshape(N), batch, tau.reshape(16), X, Xt, nn,
                      pos.reshape(N * 3), template.reshape(T * 3))
    p3 = pos.T.reshape(3, NR, 128)
    out3 = _solve(G, H, p3, batch.reshape(NR, 128))
    return out3.reshape(3, N).T
